# baseline (speedup 1.0000x reference)
import functools

import jax
import jax.numpy as jnp
from jax import lax
from jax.experimental import pallas as pl
from jax.experimental.pallas import tpu as pltpu
from jax.experimental.pallas import tpu_sc as plsc

_NC, _NS, _L = 2, 16, 16
_NW = _NC * _NS
_B = 16384
_PER_W = _B // _NW
_C = 128
_NCHUNK = _PER_W // _C
_D = 64
_W = 128
_MARGIN = 1.0

_N = 1000000
_NPAIR = _N // 2
_NB = _N // 128
_TPB = _NB // _NW + 1
_TAIL0 = _NB * 128
_TAILW = _NB % _NW


def _rsqrt(x):
    i = lax.bitcast_convert_type(x, jnp.int32)
    i = jnp.int32(0x5F3759DF) - lax.shift_right_arithmetic(i, 1)
    y = lax.bitcast_convert_type(i, jnp.float32)
    for _ in range(3):
        y = y * (1.5 - 0.5 * x * y * y)
    return y


def _pack_rows(src, dst, nrows, iota):
    def prow(p, carry):
        for m in range(8):
            h = m // 4
            j0 = (16 * m) % 64
            col = jnp.full((_L,), 2 * p + h, jnp.int32)
            v = plsc.load_gather(src, [j0 + iota, col])
            dst[p, pl.ds(16 * m, 16)] = v
        return carry

    lax.fori_loop(0, nrows, prow, 0)


def _pack_body(nodeT, packed, tbuf, ttail, obuf, sem):
    wid = lax.axis_index("s") * _NC + lax.axis_index("c")
    iota = lax.iota(jnp.int32, _L)

    def step(t, carry):
        b = wid + t * _NW

        @pl.when(b < _NB)
        def _():
            c0 = pl.multiple_of(b * 128, 128)
            pltpu.async_copy(nodeT.at[:, pl.ds(c0, 128)], tbuf, sem).wait()
            _pack_rows(tbuf, obuf, 64, iota)
            pltpu.async_copy(obuf, packed.at[pl.ds(b * 64, 64)], sem).wait()

        return carry

    lax.fori_loop(0, _TPB, step, 0)

    @pl.when(wid == _TAILW)
    def _():
        pltpu.async_copy(nodeT.at[:, pl.ds(_TAIL0, 64)], ttail, sem).wait()
        _pack_rows(ttail, obuf, 32, iota)
        pltpu.async_copy(obuf.at[pl.ds(0, 32)],
                         packed.at[pl.ds(_TAIL0 // 2, 32)], sem).wait()


def _sc_body(sp, tp, sn, tn, hsp, htp, hsn, htn, r, node, link, norm, out,
             isp, itp, isn, itn, ihsp, ihtp, ihsn, ihtn, ir,
             bsp, btp, bsn, btn, brm, bw, accv, sem):
    wid = lax.axis_index("s") * _NC + lax.axis_index("c")
    base = wid * _PER_W
    iota = lax.iota(jnp.int32, _L)
    acc = jnp.zeros((_L,), jnp.float32)
    for c in range(_NCHUNK):
        off = pl.multiple_of(base + c * _C, _C)
        pltpu.sync_copy(sp.at[pl.ds(off, _C)], isp)
        pltpu.sync_copy(tp.at[pl.ds(off, _C)], itp)
        pltpu.sync_copy(sn.at[pl.ds(off, _C)], isn)
        pltpu.sync_copy(tn.at[pl.ds(off, _C)], itn)
        pltpu.sync_copy(hsp.at[pl.ds(off, _C)], ihsp)
        pltpu.sync_copy(htp.at[pl.ds(off, _C)], ihtp)
        pltpu.sync_copy(hsn.at[pl.ds(off, _C)], ihsn)
        pltpu.sync_copy(htn.at[pl.ds(off, _C)], ihtn)
        pltpu.sync_copy(r.at[pl.ds(off, _C)], ir)
        cps = [
            pltpu.async_copy(node.at[ihsp], bsp, sem),
            pltpu.async_copy(node.at[ihtp], btp, sem),
            pltpu.async_copy(node.at[ihsn], bsn, sem),
            pltpu.async_copy(node.at[ihtn], btn, sem),
            pltpu.async_copy(link.at[ir], brm, sem),
            pltpu.async_copy(norm.at[ir], bw, sem),
        ]
        for cp in cps:
            cp.wait()

        def gbody(g, acc):
            rows = iota + g * _L
            gsl = pl.ds(g * _L, _L)
            osp = (isp[gsl] & 1) * _D
            otp = (itp[gsl] & 1) * _D
            osn = (isn[gsl] & 1) * _D
            otn = (itn[gsl] & 1) * _D

            def jbody(j, carry):
                qp, mp, qn, mn, ww, rw = carry
                jb = jnp.full((_L,), j, jnp.int32)
                vsp = plsc.load_gather(bsp, [rows, osp + jb])
                vtp = plsc.load_gather(btp, [rows, otp + jb])
                vsn = plsc.load_gather(bsn, [rows, osn + jb])
                vtn = plsc.load_gather(btn, [rows, otn + jb])
                vr = plsc.load_gather(brm, [rows, jb])
                vw = plsc.load_gather(bw, [rows, jb])
                bp = vsp - vtp + vr
                bn = vsn - vtn + vr
                return (qp + bp * bp, mp + bp * vw,
                        qn + bn * bn, mn + bn * vw,
                        ww + vw * vw, rw + vr * vw)

            z = jnp.zeros((_L,), jnp.float32)
            qp, mp, qn, mn, ww, rw = lax.fori_loop(
                0, _D, jbody, (z, z, z, z, z, z))
            cfp = (mp - rw) / ww
            cfn = (mn - rw) / ww
            ddp = qp - 2.0 * cfp * mp + cfp * cfp * ww
            ddn = qn - 2.0 * cfn * mn + cfn * cfn * ww
            ddp = jnp.maximum(ddp, 1e-20)
            ddn = jnp.maximum(ddn, 1e-20)
            dp = ddp * _rsqrt(ddp)
            dn = ddn * _rsqrt(ddn)
            return acc + jnp.maximum(0.0, dp - dn + _MARGIN)

        acc = lax.fori_loop(0, _C // _L, gbody, acc)
    accv[...] = acc
    pltpu.sync_copy(accv, out.at[pl.ds(wid * _L, _L)])


_mesh = plsc.VectorSubcoreMesh(core_axis_name="c", subcore_axis_name="s")
_params = pltpu.CompilerParams(
    needs_layout_passes=False, use_tc_tiling_on_sc=True)

_pack_kernel = pl.kernel(
    _pack_body,
    out_type=jax.ShapeDtypeStruct((_NPAIR, _W), jnp.float32),
    mesh=_mesh,
    compiler_params=_params,
    scratch_types=[
        pltpu.VMEM((_D, 128), jnp.float32),
        pltpu.VMEM((_D, 64), jnp.float32),
        pltpu.VMEM((64, _W), jnp.float32),
        pltpu.SemaphoreType.DMA,
    ],
)

_sc_kernel = pl.kernel(
    _sc_body,
    out_type=jax.ShapeDtypeStruct((_NW * _L,), jnp.float32),
    mesh=_mesh,
    compiler_params=_params,
    scratch_types=(
        [pltpu.VMEM((_C,), jnp.int32) for _ in range(9)]
        + [pltpu.VMEM((_C, _W), jnp.float32) for _ in range(6)]
        + [pltpu.VMEM((_L,), jnp.float32), pltpu.SemaphoreType.DMA]
    ),
)


def kernel(sp, tp, sn, tn, r, node_emb, link_emb, norm_vector):
    sp = sp.astype(jnp.int32)
    tp = tp.astype(jnp.int32)
    sn = sn.astype(jnp.int32)
    tn = tn.astype(jnp.int32)
    r = r.astype(jnp.int32)
    packed = _pack_kernel(node_emb.T)
    link2 = jnp.pad(link_emb, ((0, 0), (0, _W - _D)))
    norm2 = jnp.pad(norm_vector, ((0, 0), (0, _W - _D)))
    partial = _sc_kernel(sp, tp, sn, tn,
                         sp >> 1, tp >> 1, sn >> 1, tn >> 1, r,
                         packed, link2, norm2)
    return jnp.sum(partial) / _B


# speedup vs baseline: 1.2091x; 1.2091x over previous
import functools

import jax
import jax.numpy as jnp
from jax import lax
from jax.experimental import pallas as pl
from jax.experimental.pallas import tpu as pltpu
from jax.experimental.pallas import tpu_sc as plsc

_NC, _NS, _L = 2, 16, 16
_NW = _NC * _NS
_B = 16384
_PER_W = _B // _NW
_C = 128
_NCHUNK = _PER_W // _C
_D = 64
_W = 128
_MARGIN = 1.0

_N = 1000000
_NPAIR = _N // 2
_BW = 256
_PR = _BW // 2
_NBLK = _N // _BW
_T2 = (_NBLK // _NW) // 2 + 1
_TAIL0 = _NBLK * _BW
_TAILW = _NBLK % _NW


def _rsqrt(x):
    i = lax.bitcast_convert_type(x, jnp.int32)
    i = jnp.int32(0x5F3759DF) - lax.shift_right_arithmetic(i, 1)
    y = lax.bitcast_convert_type(i, jnp.float32)
    for _ in range(3):
        y = y * (1.5 - 0.5 * x * y * y)
    return y


def _pack_rows(src, dst, nrows, iota):
    def prow(p2, carry):
        for u in range(2):
            p = p2 * 2 + u
            for m in range(8):
                h = m // 4
                j0 = (16 * m) % 64
                col = jnp.full((_L,), 2 * p + h, jnp.int32)
                v = plsc.load_gather(src, [j0 + iota, col])
                dst[p, pl.ds(16 * m, 16)] = v
        return carry

    lax.fori_loop(0, nrows // 2, prow, 0)


def _pack_body(nodeT, packed, tb0, tb1, ttail, ob0, ob1, si0, si1, so0, so1):
    wid = lax.axis_index("s") * _NC + lax.axis_index("c")
    iota = lax.iota(jnp.int32, _L)
    tbs, obs, sis, sos = (tb0, tb1), (ob0, ob1), (si0, si1), (so0, so1)

    def bof(t):
        return wid + t * _NW

    def in_src(t):
        return nodeT.at[:, pl.ds(pl.multiple_of(bof(t) * _BW, _BW), _BW)]

    def out_dst(t):
        return packed.at[pl.ds(pl.multiple_of(bof(t) * _PR, _PR), _PR)]

    def start_in(t, k):
        @pl.when(bof(t) < _NBLK)
        def _():
            pltpu.async_copy(in_src(t), tbs[k], sis[k])

    start_in(0, 0)
    start_in(1, 1)

    def step(t2, carry):
        for par in range(2):
            t = t2 * 2 + par
            ok = bof(t) < _NBLK

            @pl.when((bof(t) >= 2 * _NW) & ok)
            def _():
                pltpu.make_async_copy(obs[par], out_dst(t - 2), sos[par]).wait()

            @pl.when(ok)
            def _():
                pltpu.make_async_copy(in_src(t), tbs[par], sis[par]).wait()
                _pack_rows(tbs[par], obs[par], _PR, iota)
                pltpu.async_copy(obs[par], out_dst(t), sos[par])

            start_in(t + 2, par)
        return carry

    lax.fori_loop(0, _T2, step, 0)

    def drain_par(par):
        def dstep(t2, carry):
            t = t2 * 2 + par

            @pl.when((bof(t) < _NBLK) & (bof(t + 2) >= _NBLK))
            def _():
                pltpu.make_async_copy(obs[par], out_dst(t), sos[par]).wait()

            return carry

        lax.fori_loop(0, _T2, dstep, 0)

    drain_par(0)
    drain_par(1)

    @pl.when(wid == _TAILW)
    def _():
        pltpu.async_copy(nodeT.at[:, pl.ds(_TAIL0, 64)], ttail, si0).wait()
        _pack_rows(ttail, ob0, 32, iota)
        pltpu.async_copy(ob0.at[pl.ds(0, 32)],
                         packed.at[pl.ds(_TAIL0 // 2, 32)], si0).wait()


def _sc_body(sp, tp, sn, tn, hsp, htp, hsn, htn, r, node, link, norm, out,
             isp, itp, isn, itn, ihsp, ihtp, ihsn, ihtn, ir,
             bsp, btp, bsn, btn, brm, bw, accv, sem):
    wid = lax.axis_index("s") * _NC + lax.axis_index("c")
    base = wid * _PER_W
    iota = lax.iota(jnp.int32, _L)
    acc = jnp.zeros((_L,), jnp.float32)
    for c in range(_NCHUNK):
        off = pl.multiple_of(base + c * _C, _C)
        pltpu.sync_copy(sp.at[pl.ds(off, _C)], isp)
        pltpu.sync_copy(tp.at[pl.ds(off, _C)], itp)
        pltpu.sync_copy(sn.at[pl.ds(off, _C)], isn)
        pltpu.sync_copy(tn.at[pl.ds(off, _C)], itn)
        pltpu.sync_copy(hsp.at[pl.ds(off, _C)], ihsp)
        pltpu.sync_copy(htp.at[pl.ds(off, _C)], ihtp)
        pltpu.sync_copy(hsn.at[pl.ds(off, _C)], ihsn)
        pltpu.sync_copy(htn.at[pl.ds(off, _C)], ihtn)
        pltpu.sync_copy(r.at[pl.ds(off, _C)], ir)
        cps = [
            pltpu.async_copy(node.at[ihsp], bsp, sem),
            pltpu.async_copy(node.at[ihtp], btp, sem),
            pltpu.async_copy(node.at[ihsn], bsn, sem),
            pltpu.async_copy(node.at[ihtn], btn, sem),
            pltpu.async_copy(link.at[ir], brm, sem),
            pltpu.async_copy(norm.at[ir], bw, sem),
        ]
        for cp in cps:
            cp.wait()

        def gbody(g, acc):
            rows = iota + g * _L
            gsl = pl.ds(g * _L, _L)
            osp = (isp[gsl] & 1) * _D
            otp = (itp[gsl] & 1) * _D
            osn = (isn[gsl] & 1) * _D
            otn = (itn[gsl] & 1) * _D

            def jbody(j, carry):
                qp, mp, qn, mn, ww, rw = carry
                jb = jnp.full((_L,), j, jnp.int32)
                vsp = plsc.load_gather(bsp, [rows, osp + jb])
                vtp = plsc.load_gather(btp, [rows, otp + jb])
                vsn = plsc.load_gather(bsn, [rows, osn + jb])
                vtn = plsc.load_gather(btn, [rows, otn + jb])
                vr = plsc.load_gather(brm, [rows, jb])
                vw = plsc.load_gather(bw, [rows, jb])
                bp = vsp - vtp + vr
                bn = vsn - vtn + vr
                return (qp + bp * bp, mp + bp * vw,
                        qn + bn * bn, mn + bn * vw,
                        ww + vw * vw, rw + vr * vw)

            z = jnp.zeros((_L,), jnp.float32)
            qp, mp, qn, mn, ww, rw = lax.fori_loop(
                0, _D, jbody, (z, z, z, z, z, z))
            cfp = (mp - rw) / ww
            cfn = (mn - rw) / ww
            ddp = qp - 2.0 * cfp * mp + cfp * cfp * ww
            ddn = qn - 2.0 * cfn * mn + cfn * cfn * ww
            ddp = jnp.maximum(ddp, 1e-20)
            ddn = jnp.maximum(ddn, 1e-20)
            dp = ddp * _rsqrt(ddp)
            dn = ddn * _rsqrt(ddn)
            return acc + jnp.maximum(0.0, dp - dn + _MARGIN)

        acc = lax.fori_loop(0, _C // _L, gbody, acc)
    accv[...] = acc
    pltpu.sync_copy(accv, out.at[pl.ds(wid * _L, _L)])


_mesh = plsc.VectorSubcoreMesh(core_axis_name="c", subcore_axis_name="s")
_params = pltpu.CompilerParams(
    needs_layout_passes=False, use_tc_tiling_on_sc=True)

_pack_kernel = pl.kernel(
    _pack_body,
    out_type=jax.ShapeDtypeStruct((_NPAIR, _W), jnp.float32),
    mesh=_mesh,
    compiler_params=_params,
    scratch_types=[
        pltpu.VMEM((_D, _BW), jnp.float32),
        pltpu.VMEM((_D, _BW), jnp.float32),
        pltpu.VMEM((_D, 64), jnp.float32),
        pltpu.VMEM((_PR, _W), jnp.float32),
        pltpu.VMEM((_PR, _W), jnp.float32),
        pltpu.SemaphoreType.DMA,
        pltpu.SemaphoreType.DMA,
        pltpu.SemaphoreType.DMA,
        pltpu.SemaphoreType.DMA,
    ],
)

_sc_kernel = pl.kernel(
    _sc_body,
    out_type=jax.ShapeDtypeStruct((_NW * _L,), jnp.float32),
    mesh=_mesh,
    compiler_params=_params,
    scratch_types=(
        [pltpu.VMEM((_C,), jnp.int32) for _ in range(9)]
        + [pltpu.VMEM((_C, _W), jnp.float32) for _ in range(6)]
        + [pltpu.VMEM((_L,), jnp.float32), pltpu.SemaphoreType.DMA]
    ),
)


def kernel(sp, tp, sn, tn, r, node_emb, link_emb, norm_vector):
    sp = sp.astype(jnp.int32)
    tp = tp.astype(jnp.int32)
    sn = sn.astype(jnp.int32)
    tn = tn.astype(jnp.int32)
    r = r.astype(jnp.int32)
    packed = _pack_kernel(node_emb.T)
    link2 = jnp.pad(link_emb, ((0, 0), (0, _W - _D)))
    norm2 = jnp.pad(norm_vector, ((0, 0), (0, _W - _D)))
    partial = _sc_kernel(sp, tp, sn, tn,
                         sp >> 1, tp >> 1, sn >> 1, tn >> 1, r,
                         packed, link2, norm2)
    return jnp.sum(partial) / _B


# speedup vs baseline: 2.1036x; 1.7398x over previous
import functools

import jax
import jax.numpy as jnp
from jax import lax
from jax.experimental import pallas as pl
from jax.experimental.pallas import tpu as pltpu
from jax.experimental.pallas import tpu_sc as plsc

_NC, _NS, _L = 2, 16, 16
_NW = _NC * _NS
_B = 16384
_PER_W = _B // _NW
_C = 128
_NCHUNK = _PER_W // _C
_D = 64
_W = 128
_MARGIN = 1.0

_N = 1000000
_NPAIR = _N // 2
_BW = 256
_PR = _BW // 2
_NBLK = _N // _BW
_T2 = (_NBLK // _NW) // 2 + 1
_TAIL0 = _NBLK * _BW
_TAILW = _NBLK % _NW


def _rsqrt(x):
    i = lax.bitcast_convert_type(x, jnp.int32)
    i = jnp.int32(0x5F3759DF) - lax.shift_right_arithmetic(i, 1)
    y = lax.bitcast_convert_type(i, jnp.float32)
    for _ in range(3):
        y = y * (1.5 - 0.5 * x * y * y)
    return y


def _pack_rows(src, dst, nrows, iota):
    @plsc.parallel_loop(0, nrows, 1, unroll=4)
    def prow(p):
        for m in range(8):
            h = m // 4
            j0 = (16 * m) % 64
            col = jnp.full((_L,), 2 * p + h, jnp.int32)
            v = plsc.load_gather(src, [j0 + iota, col])
            dst[p, pl.ds(16 * m, 16)] = v


def _pack_body(nodeT, packed, tb0, tb1, ttail, ob0, ob1, si0, si1, so0, so1):
    wid = lax.axis_index("s") * _NC + lax.axis_index("c")
    iota = lax.iota(jnp.int32, _L)
    tbs, obs, sis, sos = (tb0, tb1), (ob0, ob1), (si0, si1), (so0, so1)

    def bof(t):
        return wid + t * _NW

    def in_src(t):
        return nodeT.at[:, pl.ds(pl.multiple_of(bof(t) * _BW, _BW), _BW)]

    def out_dst(t):
        return packed.at[pl.ds(pl.multiple_of(bof(t) * _PR, _PR), _PR)]

    def start_in(t, k):
        @pl.when(bof(t) < _NBLK)
        def _():
            pltpu.async_copy(in_src(t), tbs[k], sis[k])

    start_in(0, 0)
    start_in(1, 1)

    def step(t2, carry):
        for par in range(2):
            t = t2 * 2 + par
            ok = bof(t) < _NBLK

            @pl.when((bof(t) >= 2 * _NW) & ok)
            def _():
                pltpu.make_async_copy(obs[par], out_dst(t - 2), sos[par]).wait()

            @pl.when(ok)
            def _():
                pltpu.make_async_copy(in_src(t), tbs[par], sis[par]).wait()
                _pack_rows(tbs[par], obs[par], _PR, iota)
                pltpu.async_copy(obs[par], out_dst(t), sos[par])

            start_in(t + 2, par)
        return carry

    lax.fori_loop(0, _T2, step, 0)

    def drain_par(par):
        def dstep(t2, carry):
            t = t2 * 2 + par

            @pl.when((bof(t) < _NBLK) & (bof(t + 2) >= _NBLK))
            def _():
                pltpu.make_async_copy(obs[par], out_dst(t), sos[par]).wait()

            return carry

        lax.fori_loop(0, _T2, dstep, 0)

    drain_par(0)
    drain_par(1)

    @pl.when(wid == _TAILW)
    def _():
        pltpu.async_copy(nodeT.at[:, pl.ds(_TAIL0, 64)], ttail, si0).wait()
        _pack_rows(ttail, ob0, 32, iota)
        pltpu.async_copy(ob0.at[pl.ds(0, 32)],
                         packed.at[pl.ds(_TAIL0 // 2, 32)], si0).wait()


def _sc_body(sp, tp, sn, tn, hsp, htp, hsn, htn, r, node, link, norm, out,
             isp, itp, isn, itn, ihsp, ihtp, ihsn, ihtn, ir,
             bsp, btp, bsn, btn, brm, bw, accv, sem):
    wid = lax.axis_index("s") * _NC + lax.axis_index("c")
    base = wid * _PER_W
    iota = lax.iota(jnp.int32, _L)
    acc = jnp.zeros((_L,), jnp.float32)
    for c in range(_NCHUNK):
        off = pl.multiple_of(base + c * _C, _C)
        pltpu.sync_copy(sp.at[pl.ds(off, _C)], isp)
        pltpu.sync_copy(tp.at[pl.ds(off, _C)], itp)
        pltpu.sync_copy(sn.at[pl.ds(off, _C)], isn)
        pltpu.sync_copy(tn.at[pl.ds(off, _C)], itn)
        pltpu.sync_copy(hsp.at[pl.ds(off, _C)], ihsp)
        pltpu.sync_copy(htp.at[pl.ds(off, _C)], ihtp)
        pltpu.sync_copy(hsn.at[pl.ds(off, _C)], ihsn)
        pltpu.sync_copy(htn.at[pl.ds(off, _C)], ihtn)
        pltpu.sync_copy(r.at[pl.ds(off, _C)], ir)
        cps = [
            pltpu.async_copy(node.at[ihsp], bsp, sem),
            pltpu.async_copy(node.at[ihtp], btp, sem),
            pltpu.async_copy(node.at[ihsn], bsn, sem),
            pltpu.async_copy(node.at[ihtn], btn, sem),
            pltpu.async_copy(link.at[ir], brm, sem),
            pltpu.async_copy(norm.at[ir], bw, sem),
        ]
        for cp in cps:
            cp.wait()

        def gbody(g, acc):
            rows = iota + g * _L
            gsl = pl.ds(g * _L, _L)
            osp = (isp[gsl] & 1) * _D
            otp = (itp[gsl] & 1) * _D
            osn = (isn[gsl] & 1) * _D
            otn = (itn[gsl] & 1) * _D

            def jbody(j, carry):
                qp, mp, qn, mn, ww, rw = carry
                jb = jnp.full((_L,), j, jnp.int32)
                vsp = plsc.load_gather(bsp, [rows, osp + jb])
                vtp = plsc.load_gather(btp, [rows, otp + jb])
                vsn = plsc.load_gather(bsn, [rows, osn + jb])
                vtn = plsc.load_gather(btn, [rows, otn + jb])
                vr = plsc.load_gather(brm, [rows, jb])
                vw = plsc.load_gather(bw, [rows, jb])
                bp = vsp - vtp + vr
                bn = vsn - vtn + vr
                return (qp + bp * bp, mp + bp * vw,
                        qn + bn * bn, mn + bn * vw,
                        ww + vw * vw, rw + vr * vw)

            z = jnp.zeros((_L,), jnp.float32)
            qp, mp, qn, mn, ww, rw = lax.fori_loop(
                0, _D, jbody, (z, z, z, z, z, z))
            cfp = (mp - rw) / ww
            cfn = (mn - rw) / ww
            ddp = qp - 2.0 * cfp * mp + cfp * cfp * ww
            ddn = qn - 2.0 * cfn * mn + cfn * cfn * ww
            ddp = jnp.maximum(ddp, 1e-20)
            ddn = jnp.maximum(ddn, 1e-20)
            dp = ddp * _rsqrt(ddp)
            dn = ddn * _rsqrt(ddn)
            return acc + jnp.maximum(0.0, dp - dn + _MARGIN)

        acc = lax.fori_loop(0, _C // _L, gbody, acc)
    accv[...] = acc
    pltpu.sync_copy(accv, out.at[pl.ds(wid * _L, _L)])


_mesh = plsc.VectorSubcoreMesh(core_axis_name="c", subcore_axis_name="s")
_params = pltpu.CompilerParams(
    needs_layout_passes=False, use_tc_tiling_on_sc=True)

_pack_kernel = pl.kernel(
    _pack_body,
    out_type=jax.ShapeDtypeStruct((_NPAIR, _W), jnp.float32),
    mesh=_mesh,
    compiler_params=_params,
    scratch_types=[
        pltpu.VMEM((_D, _BW), jnp.float32),
        pltpu.VMEM((_D, _BW), jnp.float32),
        pltpu.VMEM((_D, 64), jnp.float32),
        pltpu.VMEM((_PR, _W), jnp.float32),
        pltpu.VMEM((_PR, _W), jnp.float32),
        pltpu.SemaphoreType.DMA,
        pltpu.SemaphoreType.DMA,
        pltpu.SemaphoreType.DMA,
        pltpu.SemaphoreType.DMA,
    ],
)

_sc_kernel = pl.kernel(
    _sc_body,
    out_type=jax.ShapeDtypeStruct((_NW * _L,), jnp.float32),
    mesh=_mesh,
    compiler_params=_params,
    scratch_types=(
        [pltpu.VMEM((_C,), jnp.int32) for _ in range(9)]
        + [pltpu.VMEM((_C, _W), jnp.float32) for _ in range(6)]
        + [pltpu.VMEM((_L,), jnp.float32), pltpu.SemaphoreType.DMA]
    ),
)


def kernel(sp, tp, sn, tn, r, node_emb, link_emb, norm_vector):
    sp = sp.astype(jnp.int32)
    tp = tp.astype(jnp.int32)
    sn = sn.astype(jnp.int32)
    tn = tn.astype(jnp.int32)
    r = r.astype(jnp.int32)
    packed = _pack_kernel(node_emb.T)
    link2 = jnp.pad(link_emb, ((0, 0), (0, _W - _D)))
    norm2 = jnp.pad(norm_vector, ((0, 0), (0, _W - _D)))
    partial = _sc_kernel(sp, tp, sn, tn,
                         sp >> 1, tp >> 1, sn >> 1, tn >> 1, r,
                         packed, link2, norm2)
    return jnp.sum(partial) / _B


# speedup vs baseline: 4.1625x; 1.9788x over previous
import functools

import jax
import jax.numpy as jnp
from jax import lax
from jax.experimental import pallas as pl
from jax.experimental.pallas import tpu as pltpu
from jax.experimental.pallas import tpu_sc as plsc

_NC, _NS, _L = 2, 16, 16
_NW = _NC * _NS
_B = 16384
_PER_W = _B // _NW
_C = 64
_NCHUNK = _PER_W // _C
_D = 64
_W = 128
_MARGIN = 1.0


def _rsqrt(x):
    i = lax.bitcast_convert_type(x, jnp.int32)
    i = jnp.int32(0x5F3759DF) - lax.shift_right_arithmetic(i, 1)
    y = lax.bitcast_convert_type(i, jnp.float32)
    for _ in range(3):
        y = y * (1.5 - 0.5 * x * y * y)
    return y


def _sc_body(sp, tp, sn, tn, r, node, link, norm, out,
             isp, itp, isn, itn, ir,
             bufs0, bufs1, accv, sem0, sem1):
    wid = lax.axis_index("s") * _NC + lax.axis_index("c")
    base = wid * _PER_W
    iota = lax.iota(jnp.int32, _L)
    bufs = (bufs0, bufs1)
    sems = (sem0, sem1)

    def load_idx(c):
        off = pl.multiple_of(base + c * _C, _C)
        pltpu.sync_copy(sp.at[pl.ds(off, _C)], isp.at[c])
        pltpu.sync_copy(tp.at[pl.ds(off, _C)], itp.at[c])
        pltpu.sync_copy(sn.at[pl.ds(off, _C)], isn.at[c])
        pltpu.sync_copy(tn.at[pl.ds(off, _C)], itn.at[c])
        pltpu.sync_copy(r.at[pl.ds(off, _C)], ir.at[c])

    def fire(c, k):
        sl = pl.ds(0, _W)
        return [
            pltpu.async_copy(node.at[isp.at[c], sl], bufs[k].at[0], sems[k]),
            pltpu.async_copy(node.at[itp.at[c], sl], bufs[k].at[1], sems[k]),
            pltpu.async_copy(node.at[isn.at[c], sl], bufs[k].at[2], sems[k]),
            pltpu.async_copy(node.at[itn.at[c], sl], bufs[k].at[3], sems[k]),
            pltpu.async_copy(link.at[ir.at[c], sl], bufs[k].at[4], sems[k]),
            pltpu.async_copy(norm.at[ir.at[c], sl], bufs[k].at[5], sems[k]),
        ]

    load_idx(0)
    pend = fire(0, 0)

    acc = jnp.zeros((_L,), jnp.float32)
    for c in range(_NCHUNK):
        k = c % 2
        if c + 1 < _NCHUNK:
            load_idx(c + 1)
            nxt = fire(c + 1, 1 - k)
        else:
            nxt = None
        for cp in pend:
            cp.wait()
        pend = nxt
        bsp, btp, bsn, btn, brm, bw = (bufs[k].at[i] for i in range(6))

        def gbody(g, acc):
            rows = iota + g * _L

            def jbody(j, carry):
                qp, mp, qn, mn, ww, rw = carry
                jb = jnp.full((_L,), j, jnp.int32)
                vsp = plsc.load_gather(bsp, [rows, jb])
                vtp = plsc.load_gather(btp, [rows, jb])
                vsn = plsc.load_gather(bsn, [rows, jb])
                vtn = plsc.load_gather(btn, [rows, jb])
                vr = plsc.load_gather(brm, [rows, jb])
                vw = plsc.load_gather(bw, [rows, jb])
                bp = vsp - vtp + vr
                bn = vsn - vtn + vr
                return (qp + bp * bp, mp + bp * vw,
                        qn + bn * bn, mn + bn * vw,
                        ww + vw * vw, rw + vr * vw)

            z = jnp.zeros((_L,), jnp.float32)
            qp, mp, qn, mn, ww, rw = lax.fori_loop(
                0, _D, jbody, (z, z, z, z, z, z))
            cfp = (mp - rw) / ww
            cfn = (mn - rw) / ww
            ddp = qp - 2.0 * cfp * mp + cfp * cfp * ww
            ddn = qn - 2.0 * cfn * mn + cfn * cfn * ww
            ddp = jnp.maximum(ddp, 1e-20)
            ddn = jnp.maximum(ddn, 1e-20)
            dp = ddp * _rsqrt(ddp)
            dn = ddn * _rsqrt(ddn)
            return acc + jnp.maximum(0.0, dp - dn + _MARGIN)

        acc = lax.fori_loop(0, _C // _L, gbody, acc)
    accv[...] = acc
    pltpu.sync_copy(accv, out.at[pl.ds(wid * _L, _L)])


_mesh = plsc.VectorSubcoreMesh(core_axis_name="c", subcore_axis_name="s")

_sc_kernel = pl.kernel(
    _sc_body,
    out_type=jax.ShapeDtypeStruct((_NW * _L,), jnp.float32),
    mesh=_mesh,
    compiler_params=pltpu.CompilerParams(
        needs_layout_passes=False, use_tc_tiling_on_sc=True),
    scratch_types=[
        pltpu.VMEM((_NCHUNK, _C), jnp.int32),
        pltpu.VMEM((_NCHUNK, _C), jnp.int32),
        pltpu.VMEM((_NCHUNK, _C), jnp.int32),
        pltpu.VMEM((_NCHUNK, _C), jnp.int32),
        pltpu.VMEM((_NCHUNK, _C), jnp.int32),
        pltpu.VMEM((6, _C, _W), jnp.float32),
        pltpu.VMEM((6, _C, _W), jnp.float32),
        pltpu.VMEM((_L,), jnp.float32),
        pltpu.SemaphoreType.DMA,
        pltpu.SemaphoreType.DMA,
    ],
)


def kernel(sp, tp, sn, tn, r, node_emb, link_emb, norm_vector):
    sp = sp.astype(jnp.int32)
    tp = tp.astype(jnp.int32)
    sn = sn.astype(jnp.int32)
    tn = tn.astype(jnp.int32)
    r = r.astype(jnp.int32)
    partial = _sc_kernel(sp, tp, sn, tn, r, node_emb, link_emb, norm_vector)
    return jnp.sum(partial) / _B


# speedup vs baseline: 5.1415x; 1.2352x over previous
import functools

import jax
import jax.numpy as jnp
from jax import lax
from jax.experimental import pallas as pl
from jax.experimental.pallas import tpu as pltpu
from jax.experimental.pallas import tpu_sc as plsc

_NC, _NS, _L = 2, 16, 16
_NW = _NC * _NS
_B = 16384
_PER_W = _B // _NW
_C = 64
_NCHUNK = _PER_W // _C
_D = 64
_W = 128
_MARGIN = 1.0


def _rsqrt(x):
    i = lax.bitcast_convert_type(x, jnp.int32)
    i = jnp.int32(0x5F3759DF) - lax.shift_right_arithmetic(i, 1)
    y = lax.bitcast_convert_type(i, jnp.float32)
    for _ in range(3):
        y = y * (1.5 - 0.5 * x * y * y)
    return y


def _sc_body(sp, tp, sn, tn, r, node, link, norm, out,
             isp, itp, isn, itn, ir,
             bufs0, bufs1, accv, sem0, sem1):
    wid = lax.axis_index("s") * _NC + lax.axis_index("c")
    base = wid * _PER_W
    iota = lax.iota(jnp.int32, _L)
    bufs = (bufs0, bufs1)
    sems = (sem0, sem1)

    def load_idx(c):
        off = pl.multiple_of(base + c * _C, _C)
        pltpu.sync_copy(sp.at[pl.ds(off, _C)], isp.at[c])
        pltpu.sync_copy(tp.at[pl.ds(off, _C)], itp.at[c])
        pltpu.sync_copy(sn.at[pl.ds(off, _C)], isn.at[c])
        pltpu.sync_copy(tn.at[pl.ds(off, _C)], itn.at[c])
        pltpu.sync_copy(r.at[pl.ds(off, _C)], ir.at[c])

    def fire(c, k):
        sl = pl.ds(0, _W)
        return [
            pltpu.async_copy(node.at[isp.at[c], sl], bufs[k].at[0], sems[k]),
            pltpu.async_copy(node.at[itp.at[c], sl], bufs[k].at[1], sems[k]),
            pltpu.async_copy(node.at[isn.at[c], sl], bufs[k].at[2], sems[k]),
            pltpu.async_copy(node.at[itn.at[c], sl], bufs[k].at[3], sems[k]),
            pltpu.async_copy(link.at[ir.at[c], sl], bufs[k].at[4], sems[k]),
            pltpu.async_copy(norm.at[ir.at[c], sl], bufs[k].at[5], sems[k]),
        ]

    load_idx(0)
    pend = fire(0, 0)

    acc = jnp.zeros((_L,), jnp.float32)
    for c in range(_NCHUNK):
        k = c % 2
        if c + 1 < _NCHUNK:
            load_idx(c + 1)
            nxt = fire(c + 1, 1 - k)
        else:
            nxt = None
        for cp in pend:
            cp.wait()
        pend = nxt
        bsp, btp, bsn, btn, brm, bw = (bufs[k].at[i] for i in range(6))

        def gbody(g, acc):
            def ebody(l, carry):
                qp, mp, qn, mn, ww, rw = carry
                e = g * _L + l
                z = jnp.zeros((_L,), jnp.float32)
                vqp, vmp, vqn, vmn, vww, vrw = z, z, z, z, z, z
                for kk in range(_D // _L):
                    ksl = pl.ds(kk * _L, _L)
                    vsp = bsp[e, ksl]
                    vtp = btp[e, ksl]
                    vsn = bsn[e, ksl]
                    vtn = btn[e, ksl]
                    vr = brm[e, ksl]
                    vw = bw[e, ksl]
                    bpv = vsp - vtp + vr
                    bnv = vsn - vtn + vr
                    vqp = vqp + bpv * bpv
                    vmp = vmp + bpv * vw
                    vqn = vqn + bnv * bnv
                    vmn = vmn + bnv * vw
                    vww = vww + vw * vw
                    vrw = vrw + vr * vw
                sel = iota == l
                qp = qp + jnp.where(sel, jnp.sum(vqp), 0.0)
                mp = mp + jnp.where(sel, jnp.sum(vmp), 0.0)
                qn = qn + jnp.where(sel, jnp.sum(vqn), 0.0)
                mn = mn + jnp.where(sel, jnp.sum(vmn), 0.0)
                ww = ww + jnp.where(sel, jnp.sum(vww), 0.0)
                rw = rw + jnp.where(sel, jnp.sum(vrw), 0.0)
                return qp, mp, qn, mn, ww, rw

            z = jnp.zeros((_L,), jnp.float32)
            qp, mp, qn, mn, ww, rw = lax.fori_loop(
                0, _L, ebody, (z, z, z, z, z, z))
            cfp = (mp - rw) / ww
            cfn = (mn - rw) / ww
            ddp = qp - 2.0 * cfp * mp + cfp * cfp * ww
            ddn = qn - 2.0 * cfn * mn + cfn * cfn * ww
            ddp = jnp.maximum(ddp, 1e-20)
            ddn = jnp.maximum(ddn, 1e-20)
            dp = ddp * _rsqrt(ddp)
            dn = ddn * _rsqrt(ddn)
            return acc + jnp.maximum(0.0, dp - dn + _MARGIN)

        acc = lax.fori_loop(0, _C // _L, gbody, acc)
    accv[...] = acc
    pltpu.sync_copy(accv, out.at[pl.ds(wid * _L, _L)])


_mesh = plsc.VectorSubcoreMesh(core_axis_name="c", subcore_axis_name="s")

_sc_kernel = pl.kernel(
    _sc_body,
    out_type=jax.ShapeDtypeStruct((_NW * _L,), jnp.float32),
    mesh=_mesh,
    compiler_params=pltpu.CompilerParams(
        needs_layout_passes=False, use_tc_tiling_on_sc=True),
    scratch_types=[
        pltpu.VMEM((_NCHUNK, _C), jnp.int32),
        pltpu.VMEM((_NCHUNK, _C), jnp.int32),
        pltpu.VMEM((_NCHUNK, _C), jnp.int32),
        pltpu.VMEM((_NCHUNK, _C), jnp.int32),
        pltpu.VMEM((_NCHUNK, _C), jnp.int32),
        pltpu.VMEM((6, _C, _W), jnp.float32),
        pltpu.VMEM((6, _C, _W), jnp.float32),
        pltpu.VMEM((_L,), jnp.float32),
        pltpu.SemaphoreType.DMA,
        pltpu.SemaphoreType.DMA,
    ],
)


def kernel(sp, tp, sn, tn, r, node_emb, link_emb, norm_vector):
    sp = sp.astype(jnp.int32)
    tp = tp.astype(jnp.int32)
    sn = sn.astype(jnp.int32)
    tn = tn.astype(jnp.int32)
    r = r.astype(jnp.int32)
    partial = _sc_kernel(sp, tp, sn, tn, r, node_emb, link_emb, norm_vector)
    return jnp.sum(partial) / _B
